# Initial kernel scaffold; baseline (speedup 1.0000x reference)
import jax
import jax.numpy as jnp
from jax.experimental import pallas as pl


def kernel(x, logit_decay, log_sigma_raw, log_w_raw, ema_prob):
    raise NotImplementedError("write your pallas kernel here")


# speedup vs baseline: 64.8880x; 64.8880x over previous
import functools

import jax
import jax.numpy as jnp
from jax.experimental import pallas as pl
from jax.experimental.pallas import tpu as pltpu


def _gate_gelu_kernel(scal_ref, x_ref, rar_ref, o_ref, *, K, NB):
    x = x_ref[...]
    rar = rar_ref[...]
    sigma = scal_ref[0]
    w = scal_ref[1]

    ai = jax.lax.bitcast_convert_type(jnp.abs(x), jnp.int32)

    TT = x.shape[0]
    p = jnp.zeros((TT, 1), jnp.int32)
    for bit in range(30, 31 - NB - 1, -1):
        c = p | (1 << bit)
        n = jnp.sum((ai >= c).astype(jnp.int32), axis=1, keepdims=True)
        p = jnp.where(n >= K, c, p)

    step = 1 << (31 - NB)
    hi = (ai >= (p + step)).astype(jnp.float32)
    ge = (ai >= p).astype(jnp.float32)
    n_hi = jnp.sum(hi, axis=1, keepdims=True)
    n_ge = jnp.sum(ge, axis=1, keepdims=True)
    s_hi = jnp.sum(hi * rar, axis=1, keepdims=True)
    s_ge = jnp.sum(ge * rar, axis=1, keepdims=True)
    n_tie = jnp.maximum(n_ge - n_hi, 1.0)
    s_tie = s_ge - s_hi
    need = jnp.float32(K) - n_hi
    raw = (s_hi + need * (s_tie / n_tie)) * jnp.float32(1.0 / K)

    gate = 1.0 + w * jnp.tanh(sigma * raw)
    g = 0.5 * x * (1.0 + jax.lax.erf(x * 0.7071067811865476))
    o_ref[...] = g * gate


def kernel(x, logit_decay, log_sigma_raw, log_w_raw, ema_prob):
    B, T, D = x.shape
    K = max(1, D // 4)
    sigma = jax.nn.softplus(log_sigma_raw) + 0.01
    w = jax.nn.softplus(log_w_raw)
    scal = jnp.stack([sigma, w]).astype(jnp.float32)
    rar = (1.0 - ema_prob).astype(jnp.float32).reshape(1, D)

    BT = B * T
    x2 = x.reshape(BT, D)
    TT = 256
    grid = (BT // TT,)

    out = pl.pallas_call(
        functools.partial(_gate_gelu_kernel, K=K, NB=31),
        grid=grid,
        in_specs=[
            pl.BlockSpec(memory_space=pltpu.SMEM),
            pl.BlockSpec((TT, D), lambda i: (i, 0)),
            pl.BlockSpec((1, D), lambda i: (0, 0)),
        ],
        out_specs=pl.BlockSpec((TT, D), lambda i: (i, 0)),
        out_shape=jax.ShapeDtypeStruct((BT, D), x.dtype),
    )(scal, x2, rar)
    return out.reshape(B, T, D)
